# baseline (speedup 1.0000x reference)
import functools

import jax
import jax.numpy as jnp
import numpy as np
from jax import lax
from jax.experimental import pallas as pl
from jax.experimental.pallas import tpu as pltpu
from jax.experimental.pallas import tpu_sc as plsc

_N = 128
_M = _N * (_N + 1) // 2
_NC = 2
_NS = 16
_NW = _NC * _NS
_L = 16
_NCHUNK = _M // _L


def _sc_body(x_hbm, idx_hbm, o_hbm, xv0, xv1, ov0, ov1, idxv,
             isem0, isem1, osem0, osem1, nper):
    wid = lax.axis_index("s") * _NC + lax.axis_index("c")
    b0 = wid * nper
    xvs = (xv0, xv1)
    ovs = (ov0, ov1)
    isems = (isem0, isem1)
    osems = (osem0, osem1)

    pltpu.sync_copy(idx_hbm, idxv)

    def scatter_sample(par):
        xv = xvs[par]
        ov = ovs[par]

        def chunk_body(c, carry):
            sls = [pl.ds((4 * c + k) * _L, _L) for k in range(4)]
            avs = [idxv[sl] for sl in sls]
            xks = [xv[sl] for sl in sls]
            bids = [((a & 127) << 7) | (a >> 7) for a in avs]
            for k in range(4):
                plsc.store_scatter(ov, [avs[k]], xks[k])
                plsc.store_scatter(ov, [bids[k]], xks[k])
            return carry

        lax.fori_loop(0, _NCHUNK // 4, chunk_body, 0, unroll=False)

    pltpu.async_copy(x_hbm.at[b0], xvs[0], isems[0])
    pltpu.async_copy(x_hbm.at[b0 + 1], xvs[1], isems[1])

    def pair_body(p, carry):
        for par in range(2):
            i = 2 * p + par
            b = b0 + i
            pltpu.make_async_copy(x_hbm.at[b], xvs[par], isems[par]).wait()

            @pl.when(p >= 1)
            def _wait_out():
                pltpu.make_async_copy(ovs[par], o_hbm.at[b - 2],
                                      osems[par]).wait()

            scatter_sample(par)
            pltpu.async_copy(ovs[par], o_hbm.at[b], osems[par])

            @pl.when(p < nper // 2 - 1)
            def _next_in():
                pltpu.async_copy(x_hbm.at[b + 2], xvs[par], isems[par])

        return carry

    lax.fori_loop(0, nper // 2, pair_body, 0, unroll=False)
    pltpu.make_async_copy(ovs[0], o_hbm.at[b0], osems[0]).wait()
    pltpu.make_async_copy(ovs[1], o_hbm.at[b0], osems[1]).wait()


def _upper_targets():
    iu_r, iu_c = np.triu_indices(_N)
    return jnp.asarray((iu_r * _N + iu_c).astype(np.int32))


def kernel(input):
    b = input.shape[0]
    assert input.shape[1] == _M and b % (2 * _NW) == 0
    nper = b // _NW
    mesh = plsc.VectorSubcoreMesh(
        core_axis_name="c", subcore_axis_name="s",
        num_cores=_NC, num_subcores=_NS)
    out = pl.kernel(
        functools.partial(_sc_body, nper=nper),
        out_type=jax.ShapeDtypeStruct((b, _N * _N), input.dtype),
        mesh=mesh,
        scratch_types=[
            pltpu.VMEM((_M,), jnp.float32),
            pltpu.VMEM((_M,), jnp.float32),
            pltpu.VMEM((_N * _N,), jnp.float32),
            pltpu.VMEM((_N * _N,), jnp.float32),
            pltpu.VMEM((_M,), jnp.int32),
            pltpu.SemaphoreType.DMA,
            pltpu.SemaphoreType.DMA,
            pltpu.SemaphoreType.DMA,
            pltpu.SemaphoreType.DMA,
        ],
        compiler_params=pltpu.CompilerParams(needs_layout_passes=False),
    )(input, _upper_targets())
    return out.reshape(b, _N, _N)


# speedup vs baseline: 2.1779x; 2.1779x over previous
import functools

import jax
import jax.numpy as jnp
from jax import lax
from jax.experimental import pallas as pl
from jax.experimental.pallas import tpu as pltpu
from jax.experimental.pallas import tpu_sc as plsc

_N = 128
_M = _N * (_N + 1) // 2
_NN = _N * _N
_NC = 2
_NS = 16
_NW = _NC * _NS
_L = 16
_NB = _N // _L


def _sc_body(x_hbm, o_hbm, xab0, xab1, ov0, ov1,
             isem0, isem1, osem0, osem1, npairs):
    wid = lax.axis_index("s") * _NC + lax.axis_index("c")
    s0 = wid * (2 * npairs)
    xabs = (xab0, xab1)
    ovs = (ov0, ov1)
    isems = (isem0, isem1)
    osems = (osem0, osem1)

    jvs = [lax.iota(jnp.int32, _L) + _L * k for k in range(_NB)]
    svs = [127 * j - ((j * (j - 1)) >> 1) for j in jvs]

    def in_copy(pp, q):
        return pltpu.make_async_copy(
            x_hbm.at[pl.ds((s0 + 2 * pp) * _M, 2 * _M)],
            xabs[q].at[pl.ds(0, 2 * _M)], isems[q])

    def out_copy(pp, q):
        return pltpu.make_async_copy(
            ovs[q].at[pl.ds(0, 2 * _NN)],
            o_hbm.at[pl.ds((s0 + 2 * pp) * _NN, 2 * _NN)], osems[q])

    def expand_pair(q):
        xab = xabs[q]
        ov = ovs[q]

        for rb in range(_NB):
            def row_body(r, carry, rb=rb):
                off = 127 * r - ((r * (r - 1)) >> 1) + r
                for samp in range(2):
                    src = samp * _M
                    dstrow = samp * _NN + r * _N
                    rr = src + r
                    if rb < _NB - 1:
                        avals = [xab[pl.ds(src + off + _L * k2, _L)]
                                 for k2 in range(_NB - rb)]
                        for k2 in range(_NB - rb):
                            ov[pl.ds(dstrow + r + _L * k2, _L)] = avals[k2]
                    bidx = [svs[k] + rr for k in range(rb)]
                    bidx.append(jnp.where(jvs[rb] < r, svs[rb] + rr,
                                          (rr - 2 * r + off) + jvs[rb]))
                    bvals = [plsc.load_gather(xab, [ix]) for ix in bidx]
                    for k in range(rb + 1):
                        ov[pl.ds(dstrow + _L * k, _L)] = bvals[k]
                return carry

            lax.fori_loop(_L * rb, _L * (rb + 1), row_body, 0, unroll=False)

    in_copy(0, 0).start()
    in_copy(1, 1).start()

    def step_body(p, carry):
        for q in range(2):
            pp = 2 * p + q
            in_copy(pp, q).wait()

            @pl.when(p >= 1)
            def _wait_out():
                out_copy(pp - 2, q).wait()

            expand_pair(q)
            out_copy(pp, q).start()

            @pl.when(pp < npairs - 2)
            def _next_in():
                in_copy(pp + 2, q).start()

        return carry

    lax.fori_loop(0, npairs // 2, step_body, 0, unroll=False)
    out_copy(npairs - 2, 0).wait()
    out_copy(npairs - 1, 1).wait()


def kernel(input):
    b = input.shape[0]
    assert input.shape[1] == _M and b % (4 * _NW) == 0
    npairs = b // (2 * _NW)
    mesh = plsc.VectorSubcoreMesh(
        core_axis_name="c", subcore_axis_name="s",
        num_cores=_NC, num_subcores=_NS)
    out = pl.kernel(
        functools.partial(_sc_body, npairs=npairs),
        out_type=jax.ShapeDtypeStruct((b * _NN,), input.dtype),
        mesh=mesh,
        scratch_types=[
            pltpu.VMEM((2 * _M + _L,), jnp.float32),
            pltpu.VMEM((2 * _M + _L,), jnp.float32),
            pltpu.VMEM((2 * _NN + _L,), jnp.float32),
            pltpu.VMEM((2 * _NN + _L,), jnp.float32),
            pltpu.SemaphoreType.DMA,
            pltpu.SemaphoreType.DMA,
            pltpu.SemaphoreType.DMA,
            pltpu.SemaphoreType.DMA,
        ],
        compiler_params=pltpu.CompilerParams(needs_layout_passes=False),
    )(input.reshape(-1))
    return out.reshape(b, _N, _N)


# speedup vs baseline: 2.2524x; 1.0342x over previous
import functools

import jax
import jax.numpy as jnp
from jax import lax
from jax.experimental import pallas as pl
from jax.experimental.pallas import tpu as pltpu
from jax.experimental.pallas import tpu_sc as plsc

_N = 128
_M = _N * (_N + 1) // 2
_NN = _N * _N
_NC = 2
_NS = 16
_NW = _NC * _NS
_L = 16
_NB = _N // _L


def _sc_body(x_hbm, o_hbm, xab0, xab1, ov0, ov1,
             isem0, isem1, osem0, osem1, npairs):
    wid = lax.axis_index("s") * _NC + lax.axis_index("c")
    s0 = wid * (2 * npairs)
    xabs = (xab0, xab1)
    ovs = (ov0, ov1)
    isems = (isem0, isem1)
    osems = (osem0, osem1)

    jvs = [lax.iota(jnp.int32, _L) + _L * k for k in range(_NB)]
    svs = [127 * j - ((j * (j - 1)) >> 1) for j in jvs]

    def in_copy(pp, q):
        return pltpu.make_async_copy(
            x_hbm.at[pl.ds((s0 + 2 * pp) * _M, 2 * _M)],
            xabs[q].at[pl.ds(0, 2 * _M)], isems[q])

    def out_copy(pp, q):
        return pltpu.make_async_copy(
            ovs[q].at[pl.ds(0, 2 * _NN)],
            o_hbm.at[pl.ds((s0 + 2 * pp) * _NN, 2 * _NN)], osems[q])

    def expand_pair(q):
        xab = xabs[q]
        ov = ovs[q]

        for rb in range(_NB):
            def row_body(r, carry, rb=rb):
                off = 127 * r - ((r * (r - 1)) >> 1) + r
                stores = []
                for samp in range(2):
                    src = samp * _M
                    dstrow = samp * _NN + r * _N
                    rr = src + r
                    if rb < _NB - 1:
                        for k2 in range(_NB - rb):
                            stores.append(
                                (dstrow + r + _L * k2,
                                 xab[pl.ds(src + off + _L * k2, _L)]))
                    bidx = [svs[k] + rr for k in range(rb)]
                    bidx.append(jnp.where(jvs[rb] < r, svs[rb] + rr,
                                          (rr - 2 * r + off) + jvs[rb]))
                    for k in range(rb + 1):
                        stores.append((dstrow + _L * k,
                                       plsc.load_gather(xab, [bidx[k]])))
                for dst, val in stores:
                    ov[pl.ds(dst, _L)] = val
                return carry

            lax.fori_loop(_L * rb, _L * (rb + 1), row_body, 0, unroll=2)

    in_copy(0, 0).start()
    in_copy(1, 1).start()

    def step_body(p, carry):
        for q in range(2):
            pp = 2 * p + q
            in_copy(pp, q).wait()

            @pl.when(p >= 1)
            def _wait_out():
                out_copy(pp - 2, q).wait()

            expand_pair(q)
            out_copy(pp, q).start()

            @pl.when(pp < npairs - 2)
            def _next_in():
                in_copy(pp + 2, q).start()

        return carry

    lax.fori_loop(0, npairs // 2, step_body, 0, unroll=False)
    out_copy(npairs - 2, 0).wait()
    out_copy(npairs - 1, 1).wait()


def kernel(input):
    b = input.shape[0]
    assert input.shape[1] == _M and b % (4 * _NW) == 0
    npairs = b // (2 * _NW)
    mesh = plsc.VectorSubcoreMesh(
        core_axis_name="c", subcore_axis_name="s",
        num_cores=_NC, num_subcores=_NS)
    out = pl.kernel(
        functools.partial(_sc_body, npairs=npairs),
        out_type=jax.ShapeDtypeStruct((b * _NN,), input.dtype),
        mesh=mesh,
        scratch_types=[
            pltpu.VMEM((2 * _M + _L,), jnp.float32),
            pltpu.VMEM((2 * _M + _L,), jnp.float32),
            pltpu.VMEM((2 * _NN + _L,), jnp.float32),
            pltpu.VMEM((2 * _NN + _L,), jnp.float32),
            pltpu.SemaphoreType.DMA,
            pltpu.SemaphoreType.DMA,
            pltpu.SemaphoreType.DMA,
            pltpu.SemaphoreType.DMA,
        ],
        compiler_params=pltpu.CompilerParams(needs_layout_passes=False),
    )(input.reshape(-1))
    return out.reshape(b, _N, _N)


# speedup vs baseline: 3.1669x; 1.4060x over previous
import functools

import jax
import jax.numpy as jnp
from jax import lax
from jax.experimental import pallas as pl
from jax.experimental.pallas import tpu as pltpu
from jax.experimental.pallas import tpu_sc as plsc

_N = 128
_M = _N * (_N + 1) // 2
_NC = 2
_NS = 16
_NW = _NC * _NS
_L = 16
_NB = _N // _L


def _sc_body(x_hbm, o_hbm, xv00, xv01, xv10, xv11, ov0, ov1,
             isem0, isem1, osem0, osem1, npairs):
    wid = lax.axis_index("s") * _NC + lax.axis_index("c")
    s0 = wid * (2 * npairs)
    xvs = ((xv00, xv01), (xv10, xv11))
    ovs = (ov0, ov1)
    isems = (isem0, isem1)
    osems = (osem0, osem1)

    jvs = [lax.iota(jnp.int32, _L) + _L * k for k in range(_NB)]
    svs = [127 * j - ((j * (j - 1)) >> 1) for j in jvs]

    def in_copies(pp, q):
        b = s0 + 2 * pp
        return (pltpu.make_async_copy(x_hbm.at[b], xvs[q][0], isems[q]),
                pltpu.make_async_copy(x_hbm.at[b + 1], xvs[q][1], isems[q]))

    def out_copy(pp, q):
        return pltpu.make_async_copy(
            ovs[q], o_hbm.at[pl.ds(s0 + 2 * pp, 2)], osems[q])

    def expand_pair(q):
        ov = ovs[q]

        for rb in range(_NB):
            def row_body(r, carry, rb=rb):
                sr = 127 * r - ((r * (r - 1)) >> 1)
                stores = []
                for samp in range(2):
                    xv = xvs[q][samp]
                    for k2 in range(rb, _NB):
                        stores.append((samp, _L * k2,
                                       xv[pl.ds(sr + _L * k2, _L)]))
                    bidx = [svs[k] + r for k in range(rb)]
                    bidx.append(jnp.where(jvs[rb] < r, svs[rb] + r,
                                          sr + jvs[rb]))
                    for k in range(rb + 1):
                        stores.append((samp, _L * k,
                                       plsc.load_gather(xv, [bidx[k]])))
                for samp, col, val in stores:
                    ov[samp, r, pl.ds(col, _L)] = val
                return carry

            lax.fori_loop(_L * rb, _L * (rb + 1), row_body, 0, unroll=2)

    for cp in in_copies(0, 0) + in_copies(1, 1):
        cp.start()

    def step_body(p, carry):
        for q in range(2):
            pp = 2 * p + q
            for cp in in_copies(pp, q):
                cp.wait()

            @pl.when(p >= 1)
            def _wait_out():
                out_copy(pp - 2, q).wait()

            expand_pair(q)
            out_copy(pp, q).start()

            @pl.when(pp < npairs - 2)
            def _next_in():
                for cp in in_copies(pp + 2, q):
                    cp.start()

        return carry

    lax.fori_loop(0, npairs // 2, step_body, 0, unroll=False)
    out_copy(npairs - 2, 0).wait()
    out_copy(npairs - 1, 1).wait()


def kernel(input):
    b = input.shape[0]
    assert input.shape[1] == _M and b % (4 * _NW) == 0
    npairs = b // (2 * _NW)
    mesh = plsc.VectorSubcoreMesh(
        core_axis_name="c", subcore_axis_name="s",
        num_cores=_NC, num_subcores=_NS)
    return pl.kernel(
        functools.partial(_sc_body, npairs=npairs),
        out_type=jax.ShapeDtypeStruct((b, _N, _N), input.dtype),
        mesh=mesh,
        scratch_types=[
            pltpu.VMEM((_M,), jnp.float32),
            pltpu.VMEM((_M,), jnp.float32),
            pltpu.VMEM((_M,), jnp.float32),
            pltpu.VMEM((_M,), jnp.float32),
            pltpu.VMEM((2, _N, _N), jnp.float32),
            pltpu.VMEM((2, _N, _N), jnp.float32),
            pltpu.SemaphoreType.DMA,
            pltpu.SemaphoreType.DMA,
            pltpu.SemaphoreType.DMA,
            pltpu.SemaphoreType.DMA,
        ],
        compiler_params=pltpu.CompilerParams(needs_layout_passes=False),
    )(input)
